# baseline (speedup 1.0000x reference)
import functools

import jax
import jax.numpy as jnp
from jax import lax
from jax.experimental import pallas as pl
from jax.experimental.pallas import tpu as pltpu
from jax.experimental.pallas import tpu_sc as plsc

VOCAB = 1000000
DIM = 128
NNEG = 25
NOUT = 1 + NNEG
BATCH = 16384

NC = 2
NS = 16
NW = NC * NS
B_PER_W = BATCH // NW
CHUNK = 8
ROWS_PER_CHUNK = CHUNK * NOUT
N_CHUNKS = B_PER_W // CHUNK
LANES = 16
PART_PER_CHUNK = ROWS_PER_CHUNK * LANES


def _sc_dot_partials(t_input_ids, t_output_ids_flat, input_emb, output_emb):
  mesh = plsc.VectorSubcoreMesh(core_axis_name="c", subcore_axis_name="s")

  @functools.partial(
      pl.kernel,
      mesh=mesh,
      out_type=jax.ShapeDtypeStruct((BATCH * NOUT * LANES,), jnp.float32),
      scratch_types=[
          pltpu.VMEM((B_PER_W,), jnp.int32),
          pltpu.VMEM((B_PER_W * NOUT,), jnp.int32),
          pltpu.VMEM((2 * CHUNK, DIM), jnp.float32),
          pltpu.VMEM((2 * ROWS_PER_CHUNK, DIM), jnp.float32),
          pltpu.VMEM((2 * PART_PER_CHUNK,), jnp.float32),
          pltpu.SemaphoreType.DMA,
          pltpu.SemaphoreType.DMA,
          pltpu.SemaphoreType.DMA,
      ],
  )
  def k(iids_hbm, oids_hbm, iemb_hbm, oemb_hbm, out_hbm,
        iids_v, oids_v, xbuf, ybuf, part_v, sem_x, sem_y, sem_p):
    wid = lax.axis_index("s") * NC + lax.axis_index("c")
    b0 = wid * B_PER_W
    half = ROWS_PER_CHUNK // 2

    pltpu.sync_copy(iids_hbm.at[pl.ds(b0, B_PER_W)], iids_v)
    pltpu.sync_copy(oids_hbm.at[pl.ds(b0 * NOUT, B_PER_W * NOUT)], oids_v)

    def gathers(g, par):
      cx = pltpu.make_async_copy(
          iemb_hbm.at[iids_v.at[pl.ds(g * CHUNK, CHUNK)]],
          xbuf.at[pl.ds(par * CHUNK, CHUNK)], sem_x)
      cy0 = pltpu.make_async_copy(
          oemb_hbm.at[oids_v.at[pl.ds(g * ROWS_PER_CHUNK, half)]],
          ybuf.at[pl.ds(par * ROWS_PER_CHUNK, half)], sem_y)
      cy1 = pltpu.make_async_copy(
          oemb_hbm.at[oids_v.at[pl.ds(g * ROWS_PER_CHUNK + half, half)]],
          ybuf.at[pl.ds(par * ROWS_PER_CHUNK + half, half)], sem_y)
      return cx, cy0, cy1

    def part_store(g, par):
      return pltpu.make_async_copy(
          part_v.at[pl.ds(par * PART_PER_CHUNK, PART_PER_CHUNK)],
          out_hbm.at[pl.ds((b0 + g * CHUNK) * NOUT * LANES, PART_PER_CHUNK)],
          sem_p)

    for c in gathers(0, 0):
      c.start()
    for c in gathers(1, 1):
      c.start()

    def compute_chunk(par):
      nk = DIM // LANES

      @plsc.parallel_loop(0, CHUNK, step=1, unroll=4)
      def _(c):
        xs = [xbuf[par * CHUNK + c, pl.ds(kk * LANES, LANES)]
              for kk in range(nk)]
        for j in range(NOUT):
          row = c * NOUT + j
          m = [xs[kk] * ybuf[par * ROWS_PER_CHUNK + row,
                             pl.ds(kk * LANES, LANES)]
               for kk in range(nk)]
          while len(m) > 1:
            m = [m[t] + m[t + 1] for t in range(0, len(m) - 1, 2)] + (
                [m[-1]] if len(m) % 2 else [])
          part_v[pl.ds((par * ROWS_PER_CHUNK + row) * LANES, LANES)] = m[0]

    def body(gg, carry):
      for par in (0, 1):
        g = gg * 2 + par
        @pl.when(g >= 2)
        def _():
          part_store(g - 2, par).wait()

        cx, cy0, cy1 = gathers(g, par)
        cx.wait()
        cy0.wait()
        cy1.wait()

        compute_chunk(par)
        part_store(g, par).start()

        @pl.when(g + 2 < N_CHUNKS)
        def _():
          for c in gathers(g + 2, par):
            c.start()
      return carry

    lax.fori_loop(0, N_CHUNKS // 2, body, 0)
    part_store(N_CHUNKS - 2, 0).wait()
    part_store(N_CHUNKS - 1, 1).wait()

  return k(t_input_ids, t_output_ids_flat, input_emb, output_emb)


_TC_ROWS = BATCH * NOUT * LANES // DIM
_TC_BLOCK = 4096
_TC_GRID = _TC_ROWS // _TC_BLOCK


def _tc_bce(part_ref, out_ref):
  i = pl.program_id(0)
  x = part_ref[...]
  d = lax.broadcasted_iota(jnp.int32, (DIM, 8), 0)
  g = lax.broadcasted_iota(jnp.int32, (DIM, 8), 1)
  fold = jnp.where(d // LANES == g, 1.0, 0.0).astype(jnp.float32)
  logits = jax.lax.dot(x, fold, precision=jax.lax.Precision.HIGHEST)
  r = lax.broadcasted_iota(jnp.int32, (_TC_BLOCK, 8), 0) + i * _TC_BLOCK
  gg = lax.broadcasted_iota(jnp.int32, (_TC_BLOCK, 8), 1)
  kflat = r * 8 + gg
  tgt = jnp.where(kflat % NOUT == 0, 1.0, -1.0).astype(jnp.float32)
  terms = (jnp.maximum(logits, 0.0) - logits * tgt
           + jnp.log1p(jnp.exp(-jnp.abs(logits))))
  s = jnp.sum(terms)

  @pl.when(i == 0)
  def _():
    out_ref[0, 0] = 0.0

  out_ref[0, 0] += s


def kernel(t_input_ids, t_output_ids, input_emb, output_emb):
  iids = t_input_ids.astype(jnp.int32)
  oids = t_output_ids.astype(jnp.int32).reshape(-1)
  partials = _sc_dot_partials(iids, oids, input_emb, output_emb)
  part2d = partials.reshape(_TC_ROWS, DIM)
  loss = pl.pallas_call(
      _tc_bce,
      grid=(_TC_GRID,),
      in_specs=[pl.BlockSpec((_TC_BLOCK, DIM), lambda i: (i, 0))],
      out_specs=pl.BlockSpec(memory_space=pltpu.SMEM),
      out_shape=jax.ShapeDtypeStruct((1, 1), jnp.float32),
  )(part2d)
  return loss[0, 0]


# speedup vs baseline: 1.4898x; 1.4898x over previous
import functools

import jax
import jax.numpy as jnp
from jax import lax
from jax.experimental import pallas as pl
from jax.experimental.pallas import tpu as pltpu
from jax.experimental.pallas import tpu_sc as plsc

VOCAB = 1000000
DIM = 128
NNEG = 25
NOUT = 1 + NNEG
BATCH = 16384

NC = 2
NS = 16
NW = NC * NS
B_PER_W = BATCH // NW
CHUNK = 8
ROWS_PER_CHUNK = CHUNK * NOUT
N_CHUNKS = B_PER_W // CHUNK
LANES = 16
PART_PER_CHUNK = ROWS_PER_CHUNK * LANES


def _sc_dot_partials(t_input_ids, t_output_ids_flat, input_emb, output_emb):
  mesh = plsc.VectorSubcoreMesh(core_axis_name="c", subcore_axis_name="s")

  @functools.partial(
      pl.kernel,
      mesh=mesh,
      out_type=jax.ShapeDtypeStruct((BATCH * NOUT * LANES,), jnp.float32),
      scratch_types=[
          pltpu.VMEM((B_PER_W,), jnp.int32),
          pltpu.VMEM((B_PER_W * NOUT,), jnp.int32),
          pltpu.VMEM((2 * CHUNK, DIM), jnp.float32),
          pltpu.VMEM((2 * ROWS_PER_CHUNK, DIM), jnp.float32),
          pltpu.VMEM((2 * PART_PER_CHUNK,), jnp.float32),
          pltpu.SemaphoreType.DMA,
          pltpu.SemaphoreType.DMA,
          pltpu.SemaphoreType.DMA,
      ],
  )
  def k(iids_hbm, oids_hbm, iemb_hbm, oemb_hbm, out_hbm,
        iids_v, oids_v, xbuf, ybuf, part_v, sem_x, sem_y, sem_p):
    wid = lax.axis_index("s") * NC + lax.axis_index("c")
    b0 = wid * B_PER_W
    half = ROWS_PER_CHUNK // 2

    pltpu.sync_copy(iids_hbm.at[pl.ds(b0, B_PER_W)], iids_v)
    pltpu.sync_copy(oids_hbm.at[pl.ds(b0 * NOUT, B_PER_W * NOUT)], oids_v)

    def gathers(g, par):
      cx = pltpu.make_async_copy(
          iemb_hbm.at[iids_v.at[pl.ds(g * CHUNK, CHUNK)]],
          xbuf.at[pl.ds(par * CHUNK, CHUNK)], sem_x)
      cy0 = pltpu.make_async_copy(
          oemb_hbm.at[oids_v.at[pl.ds(g * ROWS_PER_CHUNK, half)]],
          ybuf.at[pl.ds(par * ROWS_PER_CHUNK, half)], sem_y)
      cy1 = pltpu.make_async_copy(
          oemb_hbm.at[oids_v.at[pl.ds(g * ROWS_PER_CHUNK + half, half)]],
          ybuf.at[pl.ds(par * ROWS_PER_CHUNK + half, half)], sem_y)
      return cx, cy0, cy1

    def part_store(g, par):
      return pltpu.make_async_copy(
          part_v.at[pl.ds(par * PART_PER_CHUNK, PART_PER_CHUNK)],
          out_hbm.at[pl.ds((b0 + g * CHUNK) * NOUT * LANES, PART_PER_CHUNK)],
          sem_p)

    for c in gathers(0, 0):
      c.start()
    for c in gathers(1, 1):
      c.start()

    def compute_chunk(par):
      nk = DIM // LANES

      @plsc.parallel_loop(0, CHUNK, step=1, unroll=2)
      def _(c):
        xs = [xbuf[par * CHUNK + c, pl.ds(kk * LANES, LANES)]
              for kk in range(nk)]
        for j in range(NOUT):
          row = c * NOUT + j
          m = [xs[kk] * ybuf[par * ROWS_PER_CHUNK + row,
                             pl.ds(kk * LANES, LANES)]
               for kk in range(nk)]
          while len(m) > 1:
            m = [m[t] + m[t + 1] for t in range(0, len(m) - 1, 2)] + (
                [m[-1]] if len(m) % 2 else [])
          part_v[pl.ds((par * ROWS_PER_CHUNK + row) * LANES, LANES)] = m[0]

    def body(gg, carry):
      for par in (0, 1):
        g = gg * 2 + par
        @pl.when(g >= 2)
        def _():
          part_store(g - 2, par).wait()

        cx, cy0, cy1 = gathers(g, par)
        cx.wait()
        cy0.wait()
        cy1.wait()

        compute_chunk(par)
        part_store(g, par).start()

        @pl.when(g + 2 < N_CHUNKS)
        def _():
          for c in gathers(g + 2, par):
            c.start()
      return carry

    lax.fori_loop(0, N_CHUNKS // 2, body, 0)
    part_store(N_CHUNKS - 2, 0).wait()
    part_store(N_CHUNKS - 1, 1).wait()

  return k(t_input_ids, t_output_ids_flat, input_emb, output_emb)


_TC_ROWS = BATCH * NOUT * LANES // DIM
_TC_BLOCK = 4096
_TC_GRID = _TC_ROWS // _TC_BLOCK


def _tc_bce(part_ref, out_ref):
  i = pl.program_id(0)
  x = part_ref[...]
  d = lax.broadcasted_iota(jnp.int32, (DIM, 8), 0)
  g = lax.broadcasted_iota(jnp.int32, (DIM, 8), 1)
  fold = jnp.where(d // LANES == g, 1.0, 0.0).astype(jnp.float32)
  logits = jax.lax.dot(x, fold, precision=jax.lax.Precision.HIGHEST)
  lt = jnp.transpose(logits)
  r = lax.broadcasted_iota(jnp.int32, (8, _TC_BLOCK), 1) + i * _TC_BLOCK
  gg = lax.broadcasted_iota(jnp.int32, (8, _TC_BLOCK), 0)
  kflat = r * 8 + gg
  tgt = jnp.where(kflat % NOUT == 0, 1.0, -1.0).astype(jnp.float32)
  terms = (jnp.maximum(lt, 0.0) - lt * tgt
           + jnp.log1p(jnp.exp(-jnp.abs(lt))))
  s = jnp.sum(terms)

  @pl.when(i == 0)
  def _():
    out_ref[0, 0] = 0.0

  out_ref[0, 0] += s


def kernel(t_input_ids, t_output_ids, input_emb, output_emb):
  iids = t_input_ids.astype(jnp.int32)
  oids = t_output_ids.astype(jnp.int32).reshape(-1)
  partials = _sc_dot_partials(iids, oids, input_emb, output_emb)
  part2d = partials.reshape(_TC_ROWS, DIM)
  loss = pl.pallas_call(
      _tc_bce,
      grid=(_TC_GRID,),
      in_specs=[pl.BlockSpec((_TC_BLOCK, DIM), lambda i: (i, 0))],
      out_specs=pl.BlockSpec(memory_space=pltpu.SMEM),
      out_shape=jax.ShapeDtypeStruct((1, 1), jnp.float32),
  )(part2d)
  return loss[0, 0]


# speedup vs baseline: 1.5601x; 1.0472x over previous
import functools

import jax
import jax.numpy as jnp
from jax import lax
from jax.experimental import pallas as pl
from jax.experimental.pallas import tpu as pltpu
from jax.experimental.pallas import tpu_sc as plsc

VOCAB = 1000000
DIM = 128
NNEG = 25
NOUT = 1 + NNEG
BATCH = 16384

NC = 2
NS = 16
NW = NC * NS
B_PER_W = BATCH // NW
CHUNK = 16
ROWS_PER_CHUNK = CHUNK * NOUT
N_CHUNKS = B_PER_W // CHUNK
LANES = 16
PART_PER_CHUNK = ROWS_PER_CHUNK * LANES


def _sc_dot_partials(t_input_ids, t_output_ids_flat, input_emb, output_emb):
  mesh = plsc.VectorSubcoreMesh(core_axis_name="c", subcore_axis_name="s")

  @functools.partial(
      pl.kernel,
      mesh=mesh,
      out_type=jax.ShapeDtypeStruct((BATCH * NOUT * LANES,), jnp.float32),
      scratch_types=[
          pltpu.VMEM((B_PER_W,), jnp.int32),
          pltpu.VMEM((B_PER_W * NOUT,), jnp.int32),
          pltpu.VMEM((2 * CHUNK, DIM), jnp.float32),
          pltpu.VMEM((2 * ROWS_PER_CHUNK, DIM), jnp.float32),
          pltpu.VMEM((PART_PER_CHUNK,), jnp.float32),
          pltpu.SemaphoreType.DMA,
          pltpu.SemaphoreType.DMA,
          pltpu.SemaphoreType.DMA,
      ],
  )
  def k(iids_hbm, oids_hbm, iemb_hbm, oemb_hbm, out_hbm,
        iids_v, oids_v, xbuf, ybuf, part_v, sem_x, sem_y, sem_p):
    wid = lax.axis_index("s") * NC + lax.axis_index("c")
    b0 = wid * B_PER_W
    quarter = ROWS_PER_CHUNK // 4

    pltpu.sync_copy(iids_hbm.at[pl.ds(b0, B_PER_W)], iids_v)
    pltpu.sync_copy(oids_hbm.at[pl.ds(b0 * NOUT, B_PER_W * NOUT)], oids_v)

    def gathers(g, par):
      cps = [pltpu.make_async_copy(
          iemb_hbm.at[iids_v.at[pl.ds(g * CHUNK, CHUNK)]],
          xbuf.at[pl.ds(par * CHUNK, CHUNK)], sem_x)]
      for q in range(4):
        cps.append(pltpu.make_async_copy(
            oemb_hbm.at[oids_v.at[pl.ds(g * ROWS_PER_CHUNK + q * quarter,
                                        quarter)]],
            ybuf.at[pl.ds(par * ROWS_PER_CHUNK + q * quarter, quarter)],
            sem_y))
      return cps

    def part_store(g):
      return pltpu.make_async_copy(
          part_v,
          out_hbm.at[pl.ds((b0 + g * CHUNK) * NOUT * LANES, PART_PER_CHUNK)],
          sem_p)

    for c in gathers(0, 0):
      c.start()
    for c in gathers(1, 1):
      c.start()

    def compute_chunk(par):
      nk = DIM // LANES

      @plsc.parallel_loop(0, CHUNK, step=1, unroll=2)
      def _(c):
        xs = [xbuf[par * CHUNK + c, pl.ds(kk * LANES, LANES)]
              for kk in range(nk)]
        for j in range(NOUT):
          row = c * NOUT + j
          m = [xs[kk] * ybuf[par * ROWS_PER_CHUNK + row,
                             pl.ds(kk * LANES, LANES)]
               for kk in range(nk)]
          while len(m) > 1:
            m = [m[t] + m[t + 1] for t in range(0, len(m) - 1, 2)] + (
                [m[-1]] if len(m) % 2 else [])
          part_v[pl.ds(row * LANES, LANES)] = m[0]

    def body(gg, carry):
      for par in (0, 1):
        g = gg * 2 + par
        for c in gathers(g, par):
          c.wait()

        @pl.when(g >= 1)
        def _():
          part_store(g - 1).wait()

        compute_chunk(par)
        part_store(g).start()

        @pl.when(g + 2 < N_CHUNKS)
        def _():
          for c in gathers(g + 2, par):
            c.start()
      return carry

    lax.fori_loop(0, N_CHUNKS // 2, body, 0)
    part_store(N_CHUNKS - 1).wait()

  return k(t_input_ids, t_output_ids_flat, input_emb, output_emb)


_TC_ROWS = BATCH * NOUT * LANES // DIM
_TC_BLOCK = 4096
_TC_GRID = _TC_ROWS // _TC_BLOCK


def _tc_bce(part_ref, out_ref):
  i = pl.program_id(0)
  x = part_ref[...]
  d = lax.broadcasted_iota(jnp.int32, (DIM, 8), 0)
  g = lax.broadcasted_iota(jnp.int32, (DIM, 8), 1)
  fold = jnp.where(d // LANES == g, 1.0, 0.0).astype(jnp.float32)
  logits = jax.lax.dot(x, fold, precision=jax.lax.Precision.HIGHEST)
  lt = jnp.transpose(logits)
  r = lax.broadcasted_iota(jnp.int32, (8, _TC_BLOCK), 1) + i * _TC_BLOCK
  gg = lax.broadcasted_iota(jnp.int32, (8, _TC_BLOCK), 0)
  kflat = r * 8 + gg
  tgt = jnp.where(kflat % NOUT == 0, 1.0, -1.0).astype(jnp.float32)
  terms = (jnp.maximum(lt, 0.0) - lt * tgt
           + jnp.log1p(jnp.exp(-jnp.abs(lt))))
  s = jnp.sum(terms)

  @pl.when(i == 0)
  def _():
    out_ref[0, 0] = 0.0

  out_ref[0, 0] += s


def kernel(t_input_ids, t_output_ids, input_emb, output_emb):
  iids = t_input_ids.astype(jnp.int32)
  oids = t_output_ids.astype(jnp.int32).reshape(-1)
  partials = _sc_dot_partials(iids, oids, input_emb, output_emb)
  part2d = partials.reshape(_TC_ROWS, DIM)
  loss = pl.pallas_call(
      _tc_bce,
      grid=(_TC_GRID,),
      in_specs=[pl.BlockSpec((_TC_BLOCK, DIM), lambda i: (i, 0))],
      out_specs=pl.BlockSpec(memory_space=pltpu.SMEM),
      out_shape=jax.ShapeDtypeStruct((1, 1), jnp.float32),
  )(part2d)
  return loss[0, 0]


# speedup vs baseline: 1.6964x; 1.0874x over previous
import functools

import jax
import jax.numpy as jnp
from jax import lax
from jax.experimental import pallas as pl
from jax.experimental.pallas import tpu as pltpu
from jax.experimental.pallas import tpu_sc as plsc

VOCAB = 1000000
DIM = 128
NNEG = 25
NOUT = 1 + NNEG
BATCH = 16384

NC = 2
NS = 16
NW = NC * NS
B_PER_W = BATCH // NW
CHUNK = 16
ROWS_PER_CHUNK = CHUNK * NOUT
N_CHUNKS = B_PER_W // CHUNK
LANES = 16
PART_PER_CHUNK = ROWS_PER_CHUNK * LANES


def _sc_dot_partials(t_input_ids, t_output_ids_flat, input_emb, output_emb):
  mesh = plsc.VectorSubcoreMesh(core_axis_name="c", subcore_axis_name="s")

  @functools.partial(
      pl.kernel,
      mesh=mesh,
      out_type=jax.ShapeDtypeStruct((BATCH * NOUT * LANES,), jnp.float32),
      scratch_types=[
          pltpu.VMEM((B_PER_W,), jnp.int32),
          pltpu.VMEM((B_PER_W * NOUT,), jnp.int32),
          pltpu.VMEM((2 * CHUNK, DIM), jnp.float32),
          pltpu.VMEM((2 * ROWS_PER_CHUNK, DIM), jnp.float32),
          pltpu.VMEM((PART_PER_CHUNK,), jnp.float32),
          pltpu.SemaphoreType.DMA,
          pltpu.SemaphoreType.DMA,
          pltpu.SemaphoreType.DMA,
      ],
  )
  def k(iids_hbm, oids_hbm, iemb_hbm, oemb_hbm, out_hbm,
        iids_v, oids_v, xbuf, ybuf, part_v, sem_x, sem_y, sem_p):
    wid = lax.axis_index("s") * NC + lax.axis_index("c")
    b0 = wid * B_PER_W
    quarter = ROWS_PER_CHUNK // 4

    pltpu.sync_copy(iids_hbm.at[pl.ds(b0, B_PER_W)], iids_v)
    pltpu.sync_copy(oids_hbm.at[pl.ds(b0 * NOUT, B_PER_W * NOUT)], oids_v)

    def gathers(g, par):
      cps = [pltpu.make_async_copy(
          iemb_hbm.at[iids_v.at[pl.ds(g * CHUNK, CHUNK)]],
          xbuf.at[pl.ds(par * CHUNK, CHUNK)], sem_x)]
      for q in range(4):
        cps.append(pltpu.make_async_copy(
            oemb_hbm.at[oids_v.at[pl.ds(g * ROWS_PER_CHUNK + q * quarter,
                                        quarter)]],
            ybuf.at[pl.ds(par * ROWS_PER_CHUNK + q * quarter, quarter)],
            sem_y))
      return cps

    def part_store(g):
      return pltpu.make_async_copy(
          part_v,
          out_hbm.at[pl.ds((b0 + g * CHUNK) * NOUT * LANES, PART_PER_CHUNK)],
          sem_p)

    for c in gathers(0, 0):
      c.start()
    for c in gathers(1, 1):
      c.start()

    def compute_chunk(par):
      nk = DIM // LANES

      @plsc.parallel_loop(0, CHUNK, step=1, unroll=2)
      def _(c):
        xs = [xbuf[par * CHUNK + c, pl.ds(kk * LANES, LANES)]
              for kk in range(nk)]
        for j in range(NOUT):
          row = c * NOUT + j
          m = [xs[kk] * ybuf[par * ROWS_PER_CHUNK + row,
                             pl.ds(kk * LANES, LANES)]
               for kk in range(nk)]
          while len(m) > 1:
            m = [m[t] + m[t + 1] for t in range(0, len(m) - 1, 2)] + (
                [m[-1]] if len(m) % 2 else [])
          part_v[pl.ds(row * LANES, LANES)] = m[0]

    def body(gg, carry):
      for par in (0, 1):
        g = gg * 2 + par
        for c in gathers(g, par):
          c.wait()

        @pl.when(g >= 1)
        def _():
          part_store(g - 1).wait()

        compute_chunk(par)
        part_store(g).start()

        @pl.when(g + 2 < N_CHUNKS)
        def _():
          for c in gathers(g + 2, par):
            c.start()
      return carry

    lax.fori_loop(0, N_CHUNKS // 2, body, 0)
    part_store(N_CHUNKS - 1).wait()

  return k(t_input_ids, t_output_ids_flat, input_emb, output_emb)


_TC_ROWS = BATCH * NOUT * LANES // DIM
_TC_BLOCK = 4096
_TC_GRID = _TC_ROWS // _TC_BLOCK


def _tc_bce(part_ref, out_ref):
  i = pl.program_id(0)
  x = part_ref[...]
  d = lax.broadcasted_iota(jnp.int32, (DIM, 8), 0)
  g = lax.broadcasted_iota(jnp.int32, (DIM, 8), 1)
  fold = jnp.where(d // LANES == g, 1.0, 0.0).astype(jnp.float32)
  logits = jax.lax.dot(x, fold)
  lt = jnp.transpose(logits)
  r = lax.broadcasted_iota(jnp.int32, (8, _TC_BLOCK), 1) + i * _TC_BLOCK
  gg = lax.broadcasted_iota(jnp.int32, (8, _TC_BLOCK), 0)
  kflat = r * 8 + gg
  tgt = jnp.where(kflat % NOUT == 0, 1.0, -1.0).astype(jnp.float32)
  terms = (jnp.maximum(lt, 0.0) - lt * tgt
           + jnp.log1p(jnp.exp(-jnp.abs(lt))))
  s = jnp.sum(terms)

  @pl.when(i == 0)
  def _():
    out_ref[0, 0] = 0.0

  out_ref[0, 0] += s


def kernel(t_input_ids, t_output_ids, input_emb, output_emb):
  iids = t_input_ids.astype(jnp.int32)
  oids = t_output_ids.astype(jnp.int32).reshape(-1)
  partials = _sc_dot_partials(iids, oids, input_emb, output_emb)
  part2d = partials.reshape(_TC_ROWS, DIM)
  loss = pl.pallas_call(
      _tc_bce,
      grid=(_TC_GRID,),
      in_specs=[pl.BlockSpec((_TC_BLOCK, DIM), lambda i: (i, 0))],
      out_specs=pl.BlockSpec(memory_space=pltpu.SMEM),
      out_shape=jax.ShapeDtypeStruct((1, 1), jnp.float32),
  )(part2d)
  return loss[0, 0]
